# baseline (speedup 1.0000x reference)
import functools

import jax
import jax.numpy as jnp
from jax import lax
from jax.experimental import pallas as pl
from jax.experimental.pallas import tpu as pltpu
from jax.experimental.pallas import tpu_sc as plsc

N = 10000
E = 320000
D_IN = 128
D_HID = 128
D_OUT = 64

NC = 2
NS = 16
NW = NC * NS
EPW = E // NW
B = 125
KB = EPW // B
KBC = 40
NROWS = E // B
NP = 10240
RPT = NP // NS

_MESH = plsc.VectorSubcoreMesh(core_axis_name="c", subcore_axis_name="s")


def _make_deg():
  @functools.partial(
      pl.kernel,
      mesh=_MESH,
      out_type=jax.ShapeDtypeStruct((NC, NP, D_HID), jnp.float32),
      scratch_types=[
          pltpu.VMEM((KBC, B), jnp.int32),
          pltpu.VMEM((B, D_HID), jnp.float32),
          pltpu.VMEM_SHARED((NP, D_HID), jnp.float32),
      ],
  )
  def k(dst_hbm, zeros_hbm, ones_hbm, out_hbm, dst_v, ones_v, acc_s):
    cid = lax.axis_index("c")
    sid = lax.axis_index("s")
    wid = sid * NC + cid
    pltpu.sync_copy(zeros_hbm.at[pl.ds(sid * RPT, RPT)],
                    acc_s.at[pl.ds(sid * RPT, RPT)])
    pltpu.sync_copy(ones_hbm, ones_v)
    plsc.subcore_barrier()

    def chunk(c, carry):
      pltpu.sync_copy(dst_hbm.at[pl.ds(wid * KB + c * KBC, KBC)], dst_v)

      def body(j, carry2):
        pltpu.sync_copy(ones_v, acc_s.at[dst_v.at[j]], add=True)
        return carry2

      lax.fori_loop(0, KBC, body, 0)
      return carry

    lax.fori_loop(0, KB // KBC, chunk, 0)
    plsc.subcore_barrier()
    pltpu.sync_copy(acc_s.at[pl.ds(sid * RPT, RPT)],
                    out_hbm.at[cid, pl.ds(sid * RPT, RPT)])

  return k


def _make_prop(d):
  @functools.partial(
      pl.kernel,
      mesh=_MESH,
      out_type=jax.ShapeDtypeStruct((NC, NP, d), jnp.float32),
      scratch_types=[
          pltpu.VMEM((KBC, B), jnp.int32),
          pltpu.VMEM((KBC, B), jnp.int32),
          pltpu.VMEM((2, B, d), jnp.float32),
          pltpu.VMEM_SHARED((NP, d), jnp.float32),
          pltpu.SemaphoreType.DMA,
      ],
  )
  def k(h_hbm, src_hbm, dst_hbm, zeros_hbm, out_hbm,
        src_v, dst_v, rows_v, acc_s, sem):
    cid = lax.axis_index("c")
    sid = lax.axis_index("s")
    wid = sid * NC + cid
    pltpu.sync_copy(zeros_hbm.at[pl.ds(sid * RPT, RPT)],
                    acc_s.at[pl.ds(sid * RPT, RPT)])
    plsc.subcore_barrier()

    def chunk(c, carry):
      pltpu.sync_copy(src_hbm.at[pl.ds(wid * KB + c * KBC, KBC)], src_v)
      pltpu.sync_copy(dst_hbm.at[pl.ds(wid * KB + c * KBC, KBC)], dst_v)
      pltpu.async_copy(h_hbm.at[src_v.at[0]], rows_v.at[0], sem)

      def body(j, carry2):
        p = lax.rem(j, 2)
        pltpu.make_async_copy(h_hbm.at[src_v.at[0]], rows_v.at[p],
                              sem).wait()

        @pl.when(j + 1 < KBC)
        def _():
          pltpu.async_copy(h_hbm.at[src_v.at[j + 1]], rows_v.at[1 - p], sem)

        pltpu.sync_copy(rows_v.at[p], acc_s.at[dst_v.at[j]], add=True)
        return carry2

      lax.fori_loop(0, KBC, body, 0)
      return carry

    lax.fori_loop(0, KB // KBC, chunk, 0)
    plsc.subcore_barrier()
    pltpu.sync_copy(acc_s.at[pl.ds(sid * RPT, RPT)],
                    out_hbm.at[cid, pl.ds(sid * RPT, RPT)])

  return k


_deg = _make_deg()
_prop128 = _make_prop(D_HID)

_BM = 1000


def _tc0(x, w1):
  def body(x_ref, w_ref, h_ref):
    h_ref[...] = jnp.dot(x_ref[...], w_ref[...],
                         preferred_element_type=jnp.float32)

  return pl.pallas_call(
      body,
      grid=(N // _BM,),
      in_specs=[
          pl.BlockSpec((_BM, D_IN), lambda i: (i, 0)),
          pl.BlockSpec((D_IN, D_HID), lambda i: (0, 0)),
      ],
      out_specs=pl.BlockSpec((_BM, D_HID), lambda i: (i, 0)),
      out_shape=jax.ShapeDtypeStruct((N, D_HID), jnp.float32),
  )(x, w1)


def _tc1(degp, h1):
  def body(deg_ref, h_ref, hp_ref, dinv_ref):
    deg = (deg_ref[0] + deg_ref[1])[:, :1]
    dinv = lax.rsqrt(deg + 1.0)
    hp_ref[...] = dinv * h_ref[...]
    dinv_ref[...] = dinv

  return pl.pallas_call(
      body,
      grid=(N // _BM,),
      in_specs=[
          pl.BlockSpec((NC, _BM, D_HID), lambda i: (0, i, 0)),
          pl.BlockSpec((_BM, D_HID), lambda i: (i, 0)),
      ],
      out_specs=[
          pl.BlockSpec((_BM, D_HID), lambda i: (i, 0)),
          pl.BlockSpec((_BM, 1), lambda i: (i, 0)),
      ],
      out_shape=[
          jax.ShapeDtypeStruct((N, D_HID), jnp.float32),
          jax.ShapeDtypeStruct((N, 1), jnp.float32),
      ],
  )(degp, h1)


def _tc2(acc1, h1p, dinv, b1, w2p):
  def body(acc_ref, h1p_ref, dinv_ref, b_ref, w_ref, o_ref):
    dinv = dinv_ref[...]
    t = dinv * (acc_ref[0] + acc_ref[1] + h1p_ref[...]) + b_ref[...]
    h = jnp.maximum(t, 0.0)
    o_ref[...] = dinv * jnp.dot(h, w_ref[...],
                                preferred_element_type=jnp.float32)

  return pl.pallas_call(
      body,
      grid=(N // _BM,),
      in_specs=[
          pl.BlockSpec((NC, _BM, D_HID), lambda i: (0, i, 0)),
          pl.BlockSpec((_BM, D_HID), lambda i: (i, 0)),
          pl.BlockSpec((_BM, 1), lambda i: (i, 0)),
          pl.BlockSpec((1, D_HID), lambda i: (0, 0)),
          pl.BlockSpec((D_HID, D_HID), lambda i: (0, 0)),
      ],
      out_specs=pl.BlockSpec((_BM, D_HID), lambda i: (i, 0)),
      out_shape=jax.ShapeDtypeStruct((N, D_HID), jnp.float32),
  )(acc1, h1p, dinv, b1, w2p)


def _tc3(acc2, h2p, dinv, b2):
  def body(acc_ref, h2p_ref, dinv_ref, b_ref, o_ref):
    t = dinv_ref[...] * (acc_ref[0] + acc_ref[1] + h2p_ref[...])
    o_ref[...] = t[:, :D_OUT] + b_ref[...]

  return pl.pallas_call(
      body,
      grid=(N // _BM,),
      in_specs=[
          pl.BlockSpec((NC, _BM, D_HID), lambda i: (0, i, 0)),
          pl.BlockSpec((_BM, D_HID), lambda i: (i, 0)),
          pl.BlockSpec((_BM, 1), lambda i: (i, 0)),
          pl.BlockSpec((1, D_OUT), lambda i: (0, 0)),
      ],
      out_specs=pl.BlockSpec((_BM, D_OUT), lambda i: (i, 0)),
      out_shape=jax.ShapeDtypeStruct((N, D_OUT), jnp.float32),
  )(acc2, h2p, dinv, b2)


def kernel(x, adjs, W1, b1, W2, b2):
  src2 = adjs[0].reshape(NROWS, B)
  dst2 = adjs[1].reshape(NROWS, B)
  zeros_hid = jnp.zeros((NP, D_HID), jnp.float32)
  ones_rows = jnp.ones((B, D_HID), jnp.float32)
  w2p = jnp.pad(W2, ((0, 0), (0, D_HID - D_OUT)))

  h1 = _tc0(x, W1)
  degp = _deg(dst2, zeros_hid, ones_rows)
  h1p, dinv = _tc1(degp, h1)
  acc1 = _prop128(h1p, src2, dst2, zeros_hid)
  h2p = _tc2(acc1, h1p, dinv, b1.reshape(1, D_HID), w2p)
  acc2 = _prop128(h2p, src2, dst2, zeros_hid)
  return _tc3(acc2, h2p, dinv, b2.reshape(1, D_OUT))


# speedup vs baseline: 1.0628x; 1.0628x over previous
import functools

import jax
import jax.numpy as jnp
from jax import lax
from jax.experimental import pallas as pl
from jax.experimental.pallas import tpu as pltpu
from jax.experimental.pallas import tpu_sc as plsc

N = 10000
E = 320000
D_IN = 128
D_HID = 128
D_OUT = 64

NC = 2
NS = 16
NW = NC * NS
EPW = E // NW
B = 125
KB = EPW // B
KBC = 40
NROWS = E // B
NP = 10240
RPT = NP // NS

_MESH = plsc.VectorSubcoreMesh(core_axis_name="c", subcore_axis_name="s")


def _make_deg(dw):
  @functools.partial(
      pl.kernel,
      mesh=_MESH,
      out_type=jax.ShapeDtypeStruct((NC, NP, dw), jnp.float32),
      scratch_types=[
          pltpu.VMEM((KBC, B), jnp.int32),
          pltpu.VMEM((B, dw), jnp.float32),
          pltpu.VMEM_SHARED((NP, dw), jnp.float32),
      ],
  )
  def k(dst_hbm, zeros_hbm, ones_hbm, out_hbm, dst_v, ones_v, acc_s):
    cid = lax.axis_index("c")
    sid = lax.axis_index("s")
    wid = sid * NC + cid
    pltpu.sync_copy(zeros_hbm.at[pl.ds(sid * RPT, RPT)],
                    acc_s.at[pl.ds(sid * RPT, RPT)])
    pltpu.sync_copy(ones_hbm, ones_v)
    plsc.subcore_barrier()

    def chunk(c, carry):
      pltpu.sync_copy(dst_hbm.at[pl.ds(wid * KB + c * KBC, KBC)], dst_v)

      def body(j, carry2):
        pltpu.sync_copy(ones_v, acc_s.at[dst_v.at[j]], add=True)
        return carry2

      lax.fori_loop(0, KBC, body, 0)
      return carry

    lax.fori_loop(0, KB // KBC, chunk, 0)
    plsc.subcore_barrier()
    pltpu.sync_copy(acc_s.at[pl.ds(sid * RPT, RPT)],
                    out_hbm.at[cid, pl.ds(sid * RPT, RPT)])

  return k


def _make_prop(d):
  @functools.partial(
      pl.kernel,
      mesh=_MESH,
      out_type=jax.ShapeDtypeStruct((NC, NP, d), jnp.float32),
      scratch_types=[
          pltpu.VMEM((KBC, B), jnp.int32),
          pltpu.VMEM((KBC, B), jnp.int32),
          pltpu.VMEM((2, B, d), jnp.float32),
          pltpu.VMEM_SHARED((NP, d), jnp.float32),
          pltpu.SemaphoreType.DMA,
      ],
  )
  def k(h_hbm, src_hbm, dst_hbm, zeros_hbm, out_hbm,
        src_v, dst_v, rows_v, acc_s, sem):
    cid = lax.axis_index("c")
    sid = lax.axis_index("s")
    wid = sid * NC + cid
    pltpu.sync_copy(zeros_hbm.at[pl.ds(sid * RPT, RPT)],
                    acc_s.at[pl.ds(sid * RPT, RPT)])
    plsc.subcore_barrier()

    def chunk(c, carry):
      pltpu.sync_copy(src_hbm.at[pl.ds(wid * KB + c * KBC, KBC)], src_v)
      pltpu.sync_copy(dst_hbm.at[pl.ds(wid * KB + c * KBC, KBC)], dst_v)
      pltpu.async_copy(h_hbm.at[src_v.at[0]], rows_v.at[0], sem)

      def body(j, carry2):
        p = lax.rem(j, 2)
        pltpu.make_async_copy(h_hbm.at[src_v.at[0]], rows_v.at[p],
                              sem).wait()

        @pl.when(j + 1 < KBC)
        def _():
          pltpu.async_copy(h_hbm.at[src_v.at[j + 1]], rows_v.at[1 - p], sem)

        pltpu.sync_copy(rows_v.at[p], acc_s.at[dst_v.at[j]], add=True)
        return carry2

      lax.fori_loop(0, KBC, body, 0)
      return carry

    lax.fori_loop(0, KB // KBC, chunk, 0)
    plsc.subcore_barrier()
    pltpu.sync_copy(acc_s.at[pl.ds(sid * RPT, RPT)],
                    out_hbm.at[cid, pl.ds(sid * RPT, RPT)])

  return k


DW = 64
_deg = _make_deg(DW)
_prop128 = _make_prop(D_HID)

_BM = 1000


def _tc1(degp, x, w1):
  def body(deg_ref, x_ref, w_ref, h_ref, dinv_ref):
    deg = (deg_ref[0] + deg_ref[1])[:, :1]
    dinv = lax.rsqrt(deg + 1.0)
    h = jnp.dot(x_ref[...], w_ref[...], preferred_element_type=jnp.float32)
    h_ref[...] = dinv * h
    dinv_ref[...] = dinv

  return pl.pallas_call(
      body,
      grid=(N // _BM,),
      in_specs=[
          pl.BlockSpec((NC, _BM, DW), lambda i: (0, i, 0)),
          pl.BlockSpec((_BM, D_IN), lambda i: (i, 0)),
          pl.BlockSpec((D_IN, D_HID), lambda i: (0, 0)),
      ],
      out_specs=[
          pl.BlockSpec((_BM, D_HID), lambda i: (i, 0)),
          pl.BlockSpec((_BM, 1), lambda i: (i, 0)),
      ],
      out_shape=[
          jax.ShapeDtypeStruct((N, D_HID), jnp.float32),
          jax.ShapeDtypeStruct((N, 1), jnp.float32),
      ],
  )(degp, x, w1)


def _tc2(acc1, h1p, dinv, b1, w2p):
  def body(acc_ref, h1p_ref, dinv_ref, b_ref, w_ref, o_ref):
    dinv = dinv_ref[...]
    t = dinv * (acc_ref[0] + acc_ref[1] + h1p_ref[...]) + b_ref[...]
    h = jnp.maximum(t, 0.0)
    o_ref[...] = dinv * jnp.dot(h, w_ref[...],
                                preferred_element_type=jnp.float32)

  return pl.pallas_call(
      body,
      grid=(N // _BM,),
      in_specs=[
          pl.BlockSpec((NC, _BM, D_HID), lambda i: (0, i, 0)),
          pl.BlockSpec((_BM, D_HID), lambda i: (i, 0)),
          pl.BlockSpec((_BM, 1), lambda i: (i, 0)),
          pl.BlockSpec((1, D_HID), lambda i: (0, 0)),
          pl.BlockSpec((D_HID, D_HID), lambda i: (0, 0)),
      ],
      out_specs=pl.BlockSpec((_BM, D_HID), lambda i: (i, 0)),
      out_shape=jax.ShapeDtypeStruct((N, D_HID), jnp.float32),
  )(acc1, h1p, dinv, b1, w2p)


def _tc3(acc2, h2p, dinv, b2):
  def body(acc_ref, h2p_ref, dinv_ref, b_ref, o_ref):
    t = dinv_ref[...] * (acc_ref[0] + acc_ref[1] + h2p_ref[...])
    o_ref[...] = t[:, :D_OUT] + b_ref[...]

  return pl.pallas_call(
      body,
      grid=(N // _BM,),
      in_specs=[
          pl.BlockSpec((NC, _BM, D_HID), lambda i: (0, i, 0)),
          pl.BlockSpec((_BM, D_HID), lambda i: (i, 0)),
          pl.BlockSpec((_BM, 1), lambda i: (i, 0)),
          pl.BlockSpec((1, D_OUT), lambda i: (0, 0)),
      ],
      out_specs=pl.BlockSpec((_BM, D_OUT), lambda i: (i, 0)),
      out_shape=jax.ShapeDtypeStruct((N, D_OUT), jnp.float32),
  )(acc2, h2p, dinv, b2)


def kernel(x, adjs, W1, b1, W2, b2):
  src2 = adjs[0].reshape(NROWS, B)
  dst2 = adjs[1].reshape(NROWS, B)
  zeros_hid = jnp.zeros((NP, D_HID), jnp.float32)
  zeros_dw = jnp.zeros((NP, DW), jnp.float32)
  ones_rows = jnp.ones((B, DW), jnp.float32)
  w2p = jnp.pad(W2, ((0, 0), (0, D_HID - D_OUT)))

  degp = _deg(dst2, zeros_dw, ones_rows)
  h1p, dinv = _tc1(degp, x, W1)
  acc1 = _prop128(h1p, src2, dst2, zeros_hid)
  h2p = _tc2(acc1, h1p, dinv, b1.reshape(1, D_HID), w2p)
  acc2 = _prop128(h2p, src2, dst2, zeros_hid)
  return _tc3(acc2, h2p, dinv, b2.reshape(1, D_OUT))
